# baseline (speedup 1.0000x reference)
import functools

import jax
import jax.numpy as jnp
from jax.experimental import pallas as pl
from jax.experimental.pallas import tpu as pltpu

_B = 16384
_C = 100
_ROWS = 4096
_SCALE = -0.5 / _B


def _half(o, t):
    iota = jax.lax.broadcasted_iota(jnp.int32, o.shape, 1)
    g = jnp.where(iota == t, (1.0 + _C) * o, o)
    gsum = jnp.sum(g)
    lse = jnp.log(jnp.sum(jnp.exp(o), axis=1))
    return gsum * (1.0 / _C) - 2.0 * jnp.sum(lse)


def _loss_kernel(oa_ref, ob_ref, ta_ref, tb_ref, acc_ref):
    partial = _SCALE * (_half(oa_ref[...], ta_ref[...]) + _half(ob_ref[...], tb_ref[...]))

    @pl.when(pl.program_id(0) == 0)
    def _init():
        acc_ref[0] = 0.0

    acc_ref[0] += partial


@functools.partial(jax.jit, static_argnames=())
def _loss(outputs, targets):
    grid = (_B // 2) // _ROWS
    half = _B // 2
    t2 = targets.reshape(_B, 1)
    acc = pl.pallas_call(
        _loss_kernel,
        grid=(grid,),
        in_specs=[
            pl.BlockSpec((_ROWS, _C), lambda i: (i, 0)),
            pl.BlockSpec((_ROWS, _C), lambda i, h=half // _ROWS: (i + h, 0)),
            pl.BlockSpec((_ROWS, 1), lambda i: (i, 0)),
            pl.BlockSpec((_ROWS, 1), lambda i, h=half // _ROWS: (i + h, 0)),
        ],
        out_specs=pl.BlockSpec(memory_space=pltpu.SMEM),
        out_shape=jax.ShapeDtypeStruct((1,), jnp.float32),
    )(outputs, outputs, t2, t2)
    return acc[0]


def kernel(outputs, targets, epoch, indexs, ema):
    return _loss(outputs, targets)


# speedup vs baseline: 1.3284x; 1.3284x over previous
import functools

import jax
import jax.numpy as jnp
from jax.experimental import pallas as pl
from jax.experimental.pallas import tpu as pltpu

_B = 16384
_C = 100
_ROWS = 8192
_SCALE = -0.5 / _B


def _loss_kernel(out_ref, tgt_ref, acc_ref):
    o = out_ref[...]
    tt = tgt_ref[...]
    t8 = tt[0].astype(jnp.int8)
    t = jnp.transpose(t8).astype(jnp.float32)
    iota = jax.lax.broadcasted_iota(jnp.int32, o.shape, 1).astype(jnp.float32)
    g = jnp.where(iota == t, (1.0 + _C) * o, o)
    gsum = jnp.sum(g)
    lse = jnp.log(jnp.sum(jnp.exp(o), axis=1))
    partial = _SCALE * (gsum * (1.0 / _C) - 2.0 * jnp.sum(lse))

    @pl.when(pl.program_id(0) == 0)
    def _init():
        acc_ref[0] = 0.0

    acc_ref[0] += partial


@functools.partial(jax.jit, static_argnames=())
def _loss(outputs, targets):
    grid = _B // _ROWS
    acc = pl.pallas_call(
        _loss_kernel,
        grid=(grid,),
        in_specs=[
            pl.BlockSpec((_ROWS, _C), lambda i: (i, 0)),
            pl.BlockSpec((1, 1, _ROWS), lambda i: (i, 0, 0)),
        ],
        out_specs=pl.BlockSpec(memory_space=pltpu.SMEM),
        out_shape=jax.ShapeDtypeStruct((1,), jnp.float32),
    )(outputs, targets.reshape(_B // _ROWS, 1, _ROWS))
    return acc[0]


def kernel(outputs, targets, epoch, indexs, ema):
    return _loss(outputs, targets)
